# baseline (speedup 1.0000x reference)
import functools

import jax
import jax.numpy as jnp
from jax import lax
from jax.experimental import pallas as pl
from jax.experimental.pallas import tpu as pltpu
from jax.experimental.pallas import tpu_sc as plsc

B, T, D = 16, 576, 256
N = B * T
K = 8192
BM = 512
BK = 2048

NC, NS = 2, 16
NW = NC * NS
ROWS_PER_W = N // NW
GCHUNK = 96
NCHUNK = ROWS_PER_W // GCHUNK


PARTS = ((0, 2736), (2736, 5472), (5472, K))


def _argmin_body(x2_ref, x_ref, w_ref, c_ref, idx_ref, loss_ref):
    i = pl.program_id(0)
    x2 = x2_ref[...]
    xs = (x_ref[...] * 2.0).astype(jnp.bfloat16)
    pmin = [None, None, None]
    pidx = [None, None, None]
    for j in range(K // BK):
        lo = j * BK
        w = w_ref[lo:lo + BK, :]
        c = c_ref[lo:lo + BK]
        t = lax.dot_general(xs, w, (((1,), (1,)), ((), ())),
                            preferred_element_type=jnp.float32)
        scores = (x2[:, None] - t) + c[None, :]
        for p, (s, e) in enumerate(PARTS):
            ov_lo, ov_hi = max(lo, s), min(lo + BK, e)
            if ov_lo >= ov_hi:
                continue
            if ov_lo == lo and ov_hi == lo + BK:
                ms = scores
            else:
                iota = lax.broadcasted_iota(jnp.int32, (BM, BK), 1) + lo
                mask = (iota >= ov_lo) & (iota < ov_hi)
                ms = jnp.where(mask, scores, jnp.inf)
            lm = jnp.min(ms, axis=1)
            li = jnp.argmin(ms, axis=1).astype(jnp.int32) + lo
            if pmin[p] is None:
                pmin[p], pidx[p] = lm, li
            else:
                better = lm < pmin[p]
                pidx[p] = jnp.where(better, li, pidx[p])
                pmin[p] = jnp.minimum(pmin[p], lm)
    accv = pmin[0].astype(jnp.bfloat16).astype(jnp.float32)
    acci = pidx[0]
    accval = pmin[0]
    for p in (1, 2):
        keep = accv <= pmin[p]
        acci = jnp.where(keep, acci, pidx[p])
        accval = jnp.where(keep, accval, pmin[p])
        accv = jnp.where(keep, accv, pmin[p]).astype(
            jnp.bfloat16).astype(jnp.float32)
    idx_ref[0, 0, :] = acci

    @pl.when(i == 0)
    def _():
        loss_ref[...] = jnp.zeros((1, 1), jnp.float32)

    loss_ref[...] += jnp.sum(accval).reshape(1, 1)


def _argmin_call(x2, flat_x, W, c):
    nb = N // BM
    return pl.pallas_call(
        _argmin_body,
        grid=(nb,),
        in_specs=[
            pl.BlockSpec((BM,), lambda i: (i,)),
            pl.BlockSpec((BM, D), lambda i: (i, 0)),
            pl.BlockSpec((K, D), lambda i: (0, 0)),
            pl.BlockSpec((K,), lambda i: (0,)),
        ],
        out_specs=[
            pl.BlockSpec((1, 1, BM), lambda i: (i, 0, 0)),
            pl.BlockSpec((1, 1), lambda i: (0, 0)),
        ],
        out_shape=[
            jax.ShapeDtypeStruct((nb, 1, BM), jnp.int32),
            jax.ShapeDtypeStruct((1, 1), jnp.float32),
        ],
    )(x2, flat_x, W, c)


def _gather_body(w_hbm, idx_hbm, out_hbm, idx_v, rows_v, sem):
    wid = lax.axis_index("s") * NC + lax.axis_index("c")
    base = wid * ROWS_PER_W
    pltpu.sync_copy(idx_hbm.at[wid], idx_v)
    for k in range(NCHUNK):
        pltpu.async_copy(w_hbm.at[idx_v.at[k]],
                         rows_v.at[pl.ds(k * GCHUNK, GCHUNK)], sem).wait()
    pltpu.sync_copy(rows_v, out_hbm.at[pl.ds(base, ROWS_PER_W)])


@functools.cache
def _make_gather():
    return pl.kernel(
        _gather_body,
        out_type=jax.ShapeDtypeStruct((N, D), jnp.float32),
        mesh=plsc.VectorSubcoreMesh(core_axis_name="c", subcore_axis_name="s"),
        scratch_types=[
            pltpu.VMEM((NCHUNK, GCHUNK), jnp.int32),
            pltpu.VMEM((ROWS_PER_W, D), jnp.float32),
            pltpu.SemaphoreType.DMA,
        ],
    )


def kernel(x, W):
    flat_x = x.reshape(N, D)
    x2 = jnp.sum(flat_x ** 2, axis=1)
    c = jnp.sum(W ** 2, axis=1)
    idx3, loss_sum = _argmin_call(x2, flat_x, W, c)
    idx = idx3.reshape(NW, NCHUNK, GCHUNK)
    q = _make_gather()(W, idx)
    quantized_st = flat_x + (q - flat_x)
    commitment_loss = loss_sum[0, 0] / float(N * D)
    return quantized_st.reshape(x.shape), commitment_loss


# speedup vs baseline: 1.1002x; 1.1002x over previous
import functools

import jax
import jax.numpy as jnp
from jax import lax
from jax.experimental import pallas as pl
from jax.experimental.pallas import tpu as pltpu
from jax.experimental.pallas import tpu_sc as plsc

B, T, D = 16, 576, 256
N = B * T
K = 8192
BM = 512
BK = 2048

NC, NS = 2, 16
NW = NC * NS
ROWS_PER_W = N // NW
GCHUNK = 96
NCHUNK = ROWS_PER_W // GCHUNK


PARTS = ((0, 2736), (2736, 5472), (5472, K))
SEGMENTS = (
    (0, 2048, 0), (2048, 2736, 0),
    (2736, 4096, 1), (4096, 5472, 1),
    (5472, 6144, 2), (6144, 8192, 2),
)


def _argmin_body(x2_ref, x_ref, w_ref, c_refs, idx_ref, loss_ref):
    i = pl.program_id(0)
    x2 = x2_ref[...]
    xs = (x_ref[...] * 2.0).astype(jnp.bfloat16)
    pmin = [None, None, None]
    pidx = [None, None, None]
    for (s, e, p), c_ref in zip(SEGMENTS, c_refs):
        w = w_ref[s:e, :]
        c = c_ref[...]
        t = lax.dot_general(xs, w, (((1,), (1,)), ((), ())),
                            preferred_element_type=jnp.float32)
        scores = (x2[:, None] - t) + c[None, :]
        lm = jnp.min(scores, axis=1)
        iota = lax.broadcasted_iota(jnp.int32, (BM, e - s), 1) + s
        li = jnp.min(jnp.where(scores == lm[:, None], iota, K), axis=1)
        if pmin[p] is None:
            pmin[p], pidx[p] = lm, li
        else:
            better = lm < pmin[p]
            pidx[p] = jnp.where(better, li, pidx[p])
            pmin[p] = jnp.minimum(pmin[p], lm)
    accv = pmin[0].astype(jnp.bfloat16).astype(jnp.float32)
    acci = pidx[0]
    accval = pmin[0]
    for p in (1, 2):
        keep = accv <= pmin[p]
        acci = jnp.where(keep, acci, pidx[p])
        accval = jnp.where(keep, accval, pmin[p])
        accv = jnp.where(keep, accv, pmin[p]).astype(
            jnp.bfloat16).astype(jnp.float32)
    idx_ref[0, 0, :] = acci

    @pl.when(i == 0)
    def _():
        loss_ref[...] = jnp.zeros((1, 1), jnp.float32)

    loss_ref[...] += jnp.sum(accval).reshape(1, 1)


def _argmin_body_wrap(x2_ref, x_ref, w_ref, c0, c1, c2, c3, c4, c5,
                      idx_ref, loss_ref):
    _argmin_body(x2_ref, x_ref, w_ref, (c0, c1, c2, c3, c4, c5),
                 idx_ref, loss_ref)


def _argmin_call(x2, flat_x, W, c):
    nb = N // BM
    c_segs = [c[s:e] for (s, e, _) in SEGMENTS]
    return pl.pallas_call(
        _argmin_body_wrap,
        grid=(nb,),
        in_specs=[
            pl.BlockSpec((BM,), lambda i: (i,)),
            pl.BlockSpec((BM, D), lambda i: (i, 0)),
            pl.BlockSpec((K, D), lambda i: (0, 0)),
        ] + [
            pl.BlockSpec((e - s,), lambda i: (0,)) for (s, e, _) in SEGMENTS
        ],
        out_specs=[
            pl.BlockSpec((1, 1, BM), lambda i: (i, 0, 0)),
            pl.BlockSpec((1, 1), lambda i: (0, 0)),
        ],
        out_shape=[
            jax.ShapeDtypeStruct((nb, 1, BM), jnp.int32),
            jax.ShapeDtypeStruct((1, 1), jnp.float32),
        ],
    )(x2, flat_x, W, *c_segs)


def _gather_body(w_hbm, idx_hbm, out_hbm, idx_v, rows_v, sem):
    wid = lax.axis_index("s") * NC + lax.axis_index("c")
    base = wid * ROWS_PER_W
    pltpu.sync_copy(idx_hbm.at[wid], idx_v)
    for k in range(NCHUNK):
        pltpu.async_copy(w_hbm.at[idx_v.at[k]],
                         rows_v.at[pl.ds(k * GCHUNK, GCHUNK)], sem).wait()
    pltpu.sync_copy(rows_v, out_hbm.at[pl.ds(base, ROWS_PER_W)])


@functools.cache
def _make_gather():
    return pl.kernel(
        _gather_body,
        out_type=jax.ShapeDtypeStruct((N, D), jnp.float32),
        mesh=plsc.VectorSubcoreMesh(core_axis_name="c", subcore_axis_name="s"),
        scratch_types=[
            pltpu.VMEM((NCHUNK, GCHUNK), jnp.int32),
            pltpu.VMEM((ROWS_PER_W, D), jnp.float32),
            pltpu.SemaphoreType.DMA,
        ],
    )


def kernel(x, W):
    flat_x = x.reshape(N, D)
    x2 = jnp.sum(flat_x ** 2, axis=1)
    c = jnp.sum(W ** 2, axis=1)
    idx3, loss_sum = _argmin_call(x2, flat_x, W, c)
    idx = idx3.reshape(NW, NCHUNK, GCHUNK)
    q = _make_gather()(W, idx)
    quantized_st = flat_x + (q - flat_x)
    commitment_loss = loss_sum[0, 0] / float(N * D)
    return quantized_st.reshape(x.shape), commitment_loss
